# baseline (speedup 1.0000x reference)
import jax
import jax.numpy as jnp
from jax.experimental import pallas as pl
from jax.experimental.pallas import tpu as pltpu

B = 16384
CC = 8192


def _body(pb_ref, ps_ref, pbm_ref, psm_ref, prm_ref, pb_out_ref, ps_out_ref):
    prm = prm_ref[...]

    d_pb = prm[0:9, :]
    d_def, d_nw, d_w = prm[9:11, :], prm[11:13, :], prm[13:15, :]
    d_h1tt, d_h1tt_off = prm[15:17, :], prm[17:19, :]
    d_h1c, d_h1c_on, d_h1c_off = prm[19:22, :], prm[22:25, :], prm[25:28, :]
    d_h2tt, d_h2tt_off = prm[28:30, :], prm[30:32, :]
    d_h2c, d_h2c_on, d_h2c_off = prm[32:34, :], prm[34:36, :], prm[36:38, :]
    d_ps = prm[38:44, :]

    for k in range(B // CC):
        sl = pl.ds(k * CC, CC)
        pb = pb_ref[:, sl]
        ps = ps_ref[:, sl]
        pbm = pbm_ref[:, sl]
        psm = psm_ref[:, sl]
        pb1 = pb * pbm + (1.0 - pbm) * d_pb

        cond_nw = (pbm[0:1, :] > 0.5) & (pb1[0:1, :] > 0.5)
        cond_w = (pbm[1:2, :] > 0.5) & (pb1[1:2, :] > 0.5)
        ht1_known = pbm[2:3, :] > 0.5
        ht1_hot = pb1[2:3, :] > 0.5
        ht1_on = ht1_known & ht1_hot
        ht1_off = ht1_known & (~ht1_hot)
        ht2_known = pbm[6:7, :] > 0.5
        ht2_hot = pb1[6:7, :] > 0.5
        ht2_on = ht2_known & ht2_hot
        ht2_off = ht2_known & (~ht2_hot)

        def_fill = jnp.where(cond_w, d_w, jnp.where(cond_nw, d_nw, d_def))
        ht1_tt = jnp.where(ht1_off, d_h1tt_off, d_h1tt)
        ht2_tt = jnp.where(ht2_off, d_h2tt_off, d_h2tt)
        ht1_cool = jnp.where(ht1_off, d_h1c_off,
                             jnp.where(ht1_on, d_h1c_on, d_h1c))
        ht2_cool = jnp.where(ht2_off, d_h2c_off,
                             jnp.where(ht2_on, d_h2c_on, d_h2c))

        pb_out_ref[0:3, sl] = pb1[0:3, :]
        m36 = pbm[3:6, :]
        pb_out_ref[3:6, sl] = pb1[3:6, :] * m36 + (1.0 - m36) * ht1_cool
        pb_out_ref[6:7, sl] = pb1[6:7, :]
        m79 = pbm[7:9, :]
        pb_out_ref[7:9, sl] = pb1[7:9, :] * m79 + (1.0 - m79) * ht2_cool

        fill_ps = jnp.concatenate([def_fill, ht1_tt, ht2_tt], axis=0)
        t = ps * psm + (1.0 - psm) * fill_ps
        ps_out_ref[:, sl] = t * psm + (1.0 - psm) * d_ps


def kernel(proc_bool, proc_scalar, proc_bool_mask, proc_scalar_mask,
           p_pb_def, p_def_def, p_def_nw, p_def_w,
           p_ht1_tt_def, p_ht1_tt_off,
           p_ht1_cool_def, p_ht1_cool_on, p_ht1_cool_off,
           p_ht2_tt_def, p_ht2_tt_off,
           p_ht2_cool_def, p_ht2_cool_on, p_ht2_cool_off, p_ps_def):
    prm = jnp.concatenate(
        [p[:, None] for p in
         (p_pb_def, p_def_def, p_def_nw, p_def_w,
          p_ht1_tt_def, p_ht1_tt_off,
          p_ht1_cool_def, p_ht1_cool_on, p_ht1_cool_off,
          p_ht2_tt_def, p_ht2_tt_off,
          p_ht2_cool_def, p_ht2_cool_on, p_ht2_cool_off, p_ps_def)],
        axis=0)

    vmem = pl.BlockSpec(memory_space=pltpu.MemorySpace.VMEM)
    pb_out, ps_out = pl.pallas_call(
        _body,
        in_specs=[vmem, vmem, vmem, vmem, vmem],
        out_specs=[vmem, vmem],
        out_shape=[jax.ShapeDtypeStruct((9, B), jnp.float32),
                   jax.ShapeDtypeStruct((6, B), jnp.float32)],
    )(proc_bool.T, proc_scalar.T, proc_bool_mask.T, proc_scalar_mask.T, prm)
    return (pb_out.T, ps_out.T)


# speedup vs baseline: 1.0736x; 1.0736x over previous
import jax
import jax.numpy as jnp
from jax.experimental import pallas as pl
from jax.experimental.pallas import tpu as pltpu

B = 16384
CC = 2048


def _body(pb_ref, ps_ref, pbm_ref, psm_ref, prm_ref, pb_out_ref, ps_out_ref):
    prm = prm_ref[...]

    d_pb = prm[0:9, :]
    d_def, d_nw, d_w = prm[9:11, :], prm[11:13, :], prm[13:15, :]
    d_h1tt, d_h1tt_off = prm[15:17, :], prm[17:19, :]
    d_h1c, d_h1c_on, d_h1c_off = prm[19:22, :], prm[22:25, :], prm[25:28, :]
    d_h2tt, d_h2tt_off = prm[28:30, :], prm[30:32, :]
    d_h2c, d_h2c_on, d_h2c_off = prm[32:34, :], prm[34:36, :], prm[36:38, :]
    d_ps = prm[38:44, :]

    for k in range(B // CC):
        sl = pl.ds(k * CC, CC)
        pb = pb_ref[:, sl]
        ps = ps_ref[:, sl]
        pbm = pbm_ref[:, sl]
        psm = psm_ref[:, sl]
        pb1 = pb * pbm + (1.0 - pbm) * d_pb

        cond_nw = (pbm[0:1, :] > 0.5) & (pb1[0:1, :] > 0.5)
        cond_w = (pbm[1:2, :] > 0.5) & (pb1[1:2, :] > 0.5)
        ht1_known = pbm[2:3, :] > 0.5
        ht1_hot = pb1[2:3, :] > 0.5
        ht1_on = ht1_known & ht1_hot
        ht1_off = ht1_known & (~ht1_hot)
        ht2_known = pbm[6:7, :] > 0.5
        ht2_hot = pb1[6:7, :] > 0.5
        ht2_on = ht2_known & ht2_hot
        ht2_off = ht2_known & (~ht2_hot)

        def_fill = jnp.where(cond_w, d_w, jnp.where(cond_nw, d_nw, d_def))
        ht1_tt = jnp.where(ht1_off, d_h1tt_off, d_h1tt)
        ht2_tt = jnp.where(ht2_off, d_h2tt_off, d_h2tt)
        ht1_cool = jnp.where(ht1_off, d_h1c_off,
                             jnp.where(ht1_on, d_h1c_on, d_h1c))
        ht2_cool = jnp.where(ht2_off, d_h2c_off,
                             jnp.where(ht2_on, d_h2c_on, d_h2c))

        pb_out_ref[0:3, sl] = pb1[0:3, :]
        m36 = pbm[3:6, :]
        pb_out_ref[3:6, sl] = pb1[3:6, :] * m36 + (1.0 - m36) * ht1_cool
        pb_out_ref[6:7, sl] = pb1[6:7, :]
        m79 = pbm[7:9, :]
        pb_out_ref[7:9, sl] = pb1[7:9, :] * m79 + (1.0 - m79) * ht2_cool

        fill_ps = jnp.concatenate([def_fill, ht1_tt, ht2_tt], axis=0)
        t = ps * psm + (1.0 - psm) * fill_ps
        ps_out_ref[:, sl] = t * psm + (1.0 - psm) * d_ps


def kernel(proc_bool, proc_scalar, proc_bool_mask, proc_scalar_mask,
           p_pb_def, p_def_def, p_def_nw, p_def_w,
           p_ht1_tt_def, p_ht1_tt_off,
           p_ht1_cool_def, p_ht1_cool_on, p_ht1_cool_off,
           p_ht2_tt_def, p_ht2_tt_off,
           p_ht2_cool_def, p_ht2_cool_on, p_ht2_cool_off, p_ps_def):
    prm = jnp.concatenate(
        [p[:, None] for p in
         (p_pb_def, p_def_def, p_def_nw, p_def_w,
          p_ht1_tt_def, p_ht1_tt_off,
          p_ht1_cool_def, p_ht1_cool_on, p_ht1_cool_off,
          p_ht2_tt_def, p_ht2_tt_off,
          p_ht2_cool_def, p_ht2_cool_on, p_ht2_cool_off, p_ps_def)],
        axis=0)

    vmem = pl.BlockSpec(memory_space=pltpu.MemorySpace.VMEM)
    pb_out, ps_out = pl.pallas_call(
        _body,
        in_specs=[vmem, vmem, vmem, vmem, vmem],
        out_specs=[vmem, vmem],
        out_shape=[jax.ShapeDtypeStruct((9, B), jnp.float32),
                   jax.ShapeDtypeStruct((6, B), jnp.float32)],
    )(proc_bool.T, proc_scalar.T, proc_bool_mask.T, proc_scalar_mask.T, prm)
    return (pb_out.T, ps_out.T)


# speedup vs baseline: 1.0951x; 1.0201x over previous
import jax
import jax.numpy as jnp
from jax.experimental import pallas as pl
from jax.experimental.pallas import tpu as pltpu

B = 16384
CB = 8192


def _body(pb_ref, ps_ref, pbm_ref, psm_ref, prm_ref, pb_out_ref, ps_out_ref):
    pb = pb_ref[...]
    ps = ps_ref[...]
    pbm = pbm_ref[...]
    psm = psm_ref[...]
    prm = prm_ref[...]

    d_pb = prm[0:9, :]
    d_def, d_nw, d_w = prm[9:11, :], prm[11:13, :], prm[13:15, :]
    d_h1tt, d_h1tt_off = prm[15:17, :], prm[17:19, :]
    d_h1c, d_h1c_on, d_h1c_off = prm[19:22, :], prm[22:25, :], prm[25:28, :]
    d_h2tt, d_h2tt_off = prm[28:30, :], prm[30:32, :]
    d_h2c, d_h2c_on, d_h2c_off = prm[32:34, :], prm[34:36, :], prm[36:38, :]
    d_ps = prm[38:44, :]

    pb1 = pbm * (pb - d_pb) + d_pb

    cond_nw = (pbm[0:1, :] > 0.5) & (pb1[0:1, :] > 0.5)
    cond_w = (pbm[1:2, :] > 0.5) & (pb1[1:2, :] > 0.5)
    ht1_known = pbm[2:3, :] > 0.5
    ht1_hot = pb1[2:3, :] > 0.5
    ht1_on = ht1_known & ht1_hot
    ht1_off = ht1_known & (~ht1_hot)
    ht2_known = pbm[6:7, :] > 0.5
    ht2_hot = pb1[6:7, :] > 0.5
    ht2_on = ht2_known & ht2_hot
    ht2_off = ht2_known & (~ht2_hot)

    def_fill = jnp.where(cond_w, d_w, jnp.where(cond_nw, d_nw, d_def))
    ht1_tt = jnp.where(ht1_off, d_h1tt_off, d_h1tt)
    ht2_tt = jnp.where(ht2_off, d_h2tt_off, d_h2tt)
    ht1_cool = jnp.where(ht1_off, d_h1c_off,
                         jnp.where(ht1_on, d_h1c_on, d_h1c))
    ht2_cool = jnp.where(ht2_off, d_h2c_off,
                         jnp.where(ht2_on, d_h2c_on, d_h2c))

    pb_out_ref[0:3, :] = pb1[0:3, :]
    m36 = pbm[3:6, :]
    pb_out_ref[3:6, :] = m36 * (pb1[3:6, :] - ht1_cool) + ht1_cool
    pb_out_ref[6:7, :] = pb1[6:7, :]
    m79 = pbm[7:9, :]
    pb_out_ref[7:9, :] = m79 * (pb1[7:9, :] - ht2_cool) + ht2_cool

    fill_ps = jnp.concatenate([def_fill, ht1_tt, ht2_tt], axis=0)
    t = psm * (ps - fill_ps) + fill_ps
    ps_out_ref[...] = psm * (t - d_ps) + d_ps


def kernel(proc_bool, proc_scalar, proc_bool_mask, proc_scalar_mask,
           p_pb_def, p_def_def, p_def_nw, p_def_w,
           p_ht1_tt_def, p_ht1_tt_off,
           p_ht1_cool_def, p_ht1_cool_on, p_ht1_cool_off,
           p_ht2_tt_def, p_ht2_tt_off,
           p_ht2_cool_def, p_ht2_cool_on, p_ht2_cool_off, p_ps_def):
    prm = jnp.concatenate(
        [p_pb_def, p_def_def, p_def_nw, p_def_w,
         p_ht1_tt_def, p_ht1_tt_off,
         p_ht1_cool_def, p_ht1_cool_on, p_ht1_cool_off,
         p_ht2_tt_def, p_ht2_tt_off,
         p_ht2_cool_def, p_ht2_cool_on, p_ht2_cool_off, p_ps_def])[:, None]

    grid = (B // CB,)
    col_spec9 = pl.BlockSpec((9, CB), lambda i: (0, i))
    col_spec6 = pl.BlockSpec((6, CB), lambda i: (0, i))
    prm_spec = pl.BlockSpec((44, 1), lambda i: (0, 0))

    pb_out, ps_out = pl.pallas_call(
        _body,
        grid=grid,
        in_specs=[col_spec9, col_spec6, col_spec9, col_spec6, prm_spec],
        out_specs=[col_spec9, col_spec6],
        out_shape=[jax.ShapeDtypeStruct((9, B), jnp.float32),
                   jax.ShapeDtypeStruct((6, B), jnp.float32)],
        compiler_params=pltpu.CompilerParams(
            dimension_semantics=("parallel",)),
    )(proc_bool.T, proc_scalar.T, proc_bool_mask.T, proc_scalar_mask.T, prm)
    return (pb_out.T, ps_out.T)
